# baseline (speedup 1.0000x reference)
import functools

import jax
import jax.numpy as jnp
from jax import lax
from jax.experimental import pallas as pl
from jax.experimental.pallas import tpu as pltpu
from jax.experimental.pallas import tpu_sc as plsc

NC = 2
NS = 16
CH = 128


def _sc_gather_body(nchunks, k_sample, s_base_nodes, s_base_nidx,
                    feat_hbm, nodes_hbm, nidx_hbm, self_out, nsum_out,
                    sidx_all, nidx_all, rows_a, acc_a, rows_b, acc_b,
                    sem_i, sem_na, sem_nb, sem_sa, sem_sb):
    wid = lax.axis_index("s") * NC + lax.axis_index("c")
    half = nchunks * CH
    w_base = wid * half
    cpi = pltpu.async_copy(
        nidx_hbm.at[pl.ds(0, k_sample), pl.ds(s_base_nidx + w_base, half)],
        nidx_all, sem_i)
    cps = pltpu.async_copy(
        nodes_hbm.at[pl.ds(s_base_nodes + w_base, half)], sidx_all, sem_i)
    cpi.wait()
    cps.wait()
    bufs = ((rows_a, acc_a, sem_na, sem_sa),
            (rows_b, acc_b, sem_nb, sem_sb))

    def pair(c, carry):
        inits, gather_state = [], []
        for i, (rows, acc, sem_n, sem_s) in enumerate(bufs):
            coff = (2 * c + i) * CH
            inits.append((
                pltpu.async_copy(
                    feat_hbm.at[nidx_all.at[0, pl.ds(coff, CH)]], acc,
                    sem_n),
                pltpu.async_copy(
                    feat_hbm.at[sidx_all.at[pl.ds(coff, CH)]], rows,
                    sem_s)))
        for i, (rows, acc, sem_n, sem_s) in enumerate(bufs):
            coff = (2 * c + i) * CH
            init, cp_s = inits[i]
            init.wait()
            adds = [pltpu.async_copy(
                        feat_hbm.at[nidx_all.at[k, pl.ds(coff, CH)]],
                        acc, sem_n, add=True)
                    for k in range(1, k_sample)]
            gather_state.append((cp_s, adds))
        for i, (rows, acc, sem_n, sem_s) in enumerate(bufs):
            coff = (2 * c + i) * CH
            cp_s, adds = gather_state[i]
            cp_s.wait()
            pltpu.sync_copy(rows, self_out.at[pl.ds(w_base + coff, CH)])
            for cp in adds:
                cp.wait()
            pltpu.sync_copy(acc, nsum_out.at[pl.ds(w_base + coff, CH)])
        return carry

    lax.fori_loop(0, nchunks // 2, pair, 0, unroll=False)


def _tc_body(s_ref, n_ref, w1_ref, w2_ref, wt_ref, o_ref, *, inv_k):
    h = (jnp.dot(s_ref[...], w1_ref[...])
         + jnp.dot(n_ref[...] * inv_k, w2_ref[...]))
    h = jnp.maximum(h, 0.0)
    o_ref[...] = jnp.dot(h, wt_ref[...])


def kernel(feat, W_agg, weight, nodes, labels, neigh_idx):
    del labels
    B = nodes.shape[0]
    K = neigh_idx.shape[1]
    D = feat.shape[1]
    C = weight.shape[0]
    NW = NC * NS
    NSPLIT = 1
    BS = B // NSPLIT
    assert BS % (NW * CH) == 0
    nchunks = BS // (NW * CH)

    mesh = plsc.VectorSubcoreMesh(
        core_axis_name="c", subcore_axis_name="s",
        num_cores=NC, num_subcores=NS)

    def make_sc(s_base_nodes, s_base_nidx):
        return pl.kernel(
            functools.partial(_sc_gather_body, nchunks, K, s_base_nodes,
                              s_base_nidx),
            out_type=(jax.ShapeDtypeStruct((BS, D), jnp.float32),
                      jax.ShapeDtypeStruct((BS, D), jnp.float32)),
            mesh=mesh,
            scratch_types=(
                [pltpu.VMEM((BS // NW,), jnp.int32),
                 pltpu.VMEM((K, BS // NW), jnp.int32)]
                + [pltpu.VMEM((CH, D), jnp.float32)] * 4
                + [pltpu.SemaphoreType.DMA] * 5),
        )

    CP = 8
    w1 = W_agg[:D]
    w2 = W_agg[D:]
    wt = jnp.zeros((D, CP), jnp.float32).at[:, :C].set(weight.T)
    bm = 4096
    tc_dense = pl.pallas_call(
        functools.partial(_tc_body, inv_k=1.0 / K),
        grid=(BS // bm,),
        in_specs=[
            pl.BlockSpec((bm, D), lambda i: (i, 0)),
            pl.BlockSpec((bm, D), lambda i: (i, 0)),
            pl.BlockSpec((D, D), lambda i: (0, 0)),
            pl.BlockSpec((D, D), lambda i: (0, 0)),
            pl.BlockSpec((D, CP), lambda i: (0, 0)),
        ],
        out_specs=pl.BlockSpec((bm, CP), lambda i: (i, 0)),
        out_shape=jax.ShapeDtypeStruct((BS, CP), jnp.float32),
    )
    nidx_t = neigh_idx.T
    outs = []
    for s in range(NSPLIT):
        self_f, nsum = make_sc(s * BS, s * BS)(feat, nodes, nidx_t)
        outs.append(tc_dense(self_f, nsum, w1, w2, wt))
    return jnp.concatenate(outs, axis=0)[:, :C]


# speedup vs baseline: 1.0104x; 1.0104x over previous
import functools

import jax
import jax.numpy as jnp
from jax import lax
from jax.experimental import pallas as pl
from jax.experimental.pallas import tpu as pltpu
from jax.experimental.pallas import tpu_sc as plsc

NC = 2
NS = 16
CH = 128


def _sc_gather_body(nchunks, k_sample, s_base_nodes, s_base_nidx,
                    feat_hbm, nodes_hbm, nidx_hbm, self_out, nsum_out,
                    sidx_all, nidx_all, rows_a, acc_a, rows_b, acc_b,
                    sem_i, sem_na, sem_nb, sem_sa, sem_sb):
    wid = lax.axis_index("s") * NC + lax.axis_index("c")
    half = nchunks * CH
    w_base = wid * half
    cpi = pltpu.async_copy(
        nidx_hbm.at[pl.ds(0, k_sample), pl.ds(s_base_nidx + w_base, half)],
        nidx_all, sem_i)
    cps = pltpu.async_copy(
        nodes_hbm.at[pl.ds(s_base_nodes + w_base, half)], sidx_all, sem_i)
    cpi.wait()
    cps.wait()
    bufs = ((rows_a, acc_a, sem_na, sem_sa),
            (rows_b, acc_b, sem_nb, sem_sb))

    def pair(c, carry):
        inits, gather_state = [], []
        for i, (rows, acc, sem_n, sem_s) in enumerate(bufs):
            coff = (2 * c + i) * CH
            inits.append((
                pltpu.async_copy(
                    feat_hbm.at[nidx_all.at[0, pl.ds(coff, CH)]], acc,
                    sem_n),
                pltpu.async_copy(
                    feat_hbm.at[sidx_all.at[pl.ds(coff, CH)]], rows,
                    sem_s)))
        for i, (rows, acc, sem_n, sem_s) in enumerate(bufs):
            coff = (2 * c + i) * CH
            init, cp_s = inits[i]
            init.wait()
            adds = [pltpu.async_copy(
                        feat_hbm.at[nidx_all.at[k, pl.ds(coff, CH)]],
                        acc, sem_n, add=True)
                    for k in range(1, k_sample)]
            gather_state.append((cp_s, adds))
        for i, (rows, acc, sem_n, sem_s) in enumerate(bufs):
            coff = (2 * c + i) * CH
            cp_s, adds = gather_state[i]
            cp_s.wait()
            pltpu.sync_copy(rows, self_out.at[pl.ds(w_base + coff, CH)])
            for cp in adds:
                cp.wait()
            pltpu.sync_copy(acc, nsum_out.at[pl.ds(w_base + coff, CH)])
        return carry

    lax.fori_loop(0, nchunks // 2, pair, 0, unroll=False)


def _tc_body(s_ref, n_ref, w1_ref, w2_ref, wt_ref, o_ref, *, inv_k):
    h = (jnp.dot(s_ref[...], w1_ref[...])
         + jnp.dot(n_ref[...] * inv_k, w2_ref[...]))
    h = jnp.maximum(h, 0.0)
    o_ref[...] = jnp.dot(h, wt_ref[...])


def kernel(feat, W_agg, weight, nodes, labels, neigh_idx):
    del labels
    B = nodes.shape[0]
    K = neigh_idx.shape[1]
    D = feat.shape[1]
    C = weight.shape[0]
    NW = NC * NS
    NSPLIT = 1
    BS = B // NSPLIT
    assert BS % (NW * CH) == 0
    nchunks = BS // (NW * CH)

    mesh = plsc.VectorSubcoreMesh(
        core_axis_name="c", subcore_axis_name="s",
        num_cores=NC, num_subcores=NS)

    def make_sc(s_base_nodes, s_base_nidx):
        return pl.kernel(
            functools.partial(_sc_gather_body, nchunks, K, s_base_nodes,
                              s_base_nidx),
            out_type=(jax.ShapeDtypeStruct((BS, D), jnp.float32),
                      jax.ShapeDtypeStruct((BS, D), jnp.float32)),
            mesh=mesh,
            scratch_types=(
                [pltpu.VMEM((BS // NW,), jnp.int32),
                 pltpu.VMEM((K, BS // NW), jnp.int32)]
                + [pltpu.VMEM((CH, D), jnp.float32)] * 4
                + [pltpu.SemaphoreType.DMA] * 5),
        )

    CP = C
    w1 = W_agg[:D]
    w2 = W_agg[D:]
    wt = weight.T
    bm = 8192
    tc_dense = pl.pallas_call(
        functools.partial(_tc_body, inv_k=1.0 / K),
        grid=(BS // bm,),
        in_specs=[
            pl.BlockSpec((bm, D), lambda i: (i, 0)),
            pl.BlockSpec((bm, D), lambda i: (i, 0)),
            pl.BlockSpec((D, D), lambda i: (0, 0)),
            pl.BlockSpec((D, D), lambda i: (0, 0)),
            pl.BlockSpec((D, CP), lambda i: (0, 0)),
        ],
        out_specs=pl.BlockSpec((bm, CP), lambda i: (i, 0)),
        out_shape=jax.ShapeDtypeStruct((BS, CP), jnp.float32),
    )
    nidx_t = neigh_idx.T
    outs = []
    for s in range(NSPLIT):
        self_f, nsum = make_sc(s * BS, s * BS)(feat, nodes, nidx_t)
        outs.append(tc_dense(self_f, nsum, w1, w2, wt))
    return jnp.concatenate(outs, axis=0) if NSPLIT > 1 else outs[0]
